# baseline (speedup 1.0000x reference)
import functools

import jax
import jax.numpy as jnp
import numpy as np
from jax.experimental import pallas as pl
from jax.experimental.pallas import tpu as pltpu
from jax.experimental.shard_map import shard_map


def _hardswish(x):
    return x * jnp.clip(x + 3.0, 0.0, 6.0) * (1.0 / 6.0)


def _kernel(C, n_pw_chunks, x_ref, bd_ref, dwb_ref, pw_ref, pwb_ref,
            o_ref, xp_ref):
    Nb, H, WC = x_ref.shape
    n_blocks = bd_ref.shape[0]
    CH = pw_ref.shape[-1]

    xp_ref[:, 0:1, :] = jnp.zeros((Nb, 1, WC), jnp.float32)
    xp_ref[:, H + 1:H + 2, :] = jnp.zeros((Nb, 1, WC), jnp.float32)

    x = x_ref[...].astype(jnp.float32).reshape(Nb * H, WC)

    for blk in range(n_blocks):
        xp_ref[:, 1:H + 1, :] = x.reshape(Nb, H, WC)

        a = xp_ref[:, 0:H, :].reshape(Nb * H, WC)
        c = xp_ref[:, 2:H + 2, :].reshape(Nb * H, WC)
        y = (jnp.dot(a, bd_ref[blk, 0], preferred_element_type=jnp.float32)
             + jnp.dot(x, bd_ref[blk, 1], preferred_element_type=jnp.float32)
             + jnp.dot(c, bd_ref[blk, 2], preferred_element_type=jnp.float32))

        y = _hardswish(y + dwb_ref[blk].reshape(1, WC))

        if n_pw_chunks == 1:
            z = jnp.dot(y, pw_ref[blk], preferred_element_type=jnp.float32)
        else:
            z = jnp.concatenate(
                [
                    jnp.dot(y[:, k * CH:(k + 1) * CH], pw_ref[blk],
                            preferred_element_type=jnp.float32)
                    for k in range(n_pw_chunks)
                ],
                axis=1,
            )
        z = _hardswish(z + pwb_ref[blk].reshape(1, WC))

        x = z + x

    o_ref[...] = x.reshape(Nb, H, WC).astype(o_ref.dtype)


def _band_masks(W, C):
    WC = W * C
    masks = np.zeros((3, WC, WC), np.float32)
    eye_c = np.eye(C, dtype=np.float32)
    for kw in range(3):
        for w in range(W):
            v = w + kw - 1
            if 0 <= v < W:
                masks[kw, v * C:(v + 1) * C, w * C:(w + 1) * C] = eye_c
    return masks


def _pw_mask(reps, C):
    m = np.zeros((reps * C, reps * C), np.float32)
    for u in range(reps):
        m[u * C:(u + 1) * C, u * C:(u + 1) * C] = 1.0
    return m


def _build_params(dww, dwb, pww, pwb, W):
    n, _, _, C = dww.shape
    WC = W * C
    dww_f = jnp.tile(dww[:, :, :, None, :], (1, 1, 1, W, 1)).reshape(n, 3, 3, WC)
    w_idx = np.arange(WC) // C
    edge = np.ones((3, WC), np.float32)
    edge[0] = (w_idx >= 1).astype(np.float32)
    edge[2] = (w_idx <= W - 2).astype(np.float32)
    dww_f = dww_f * edge[None, None]
    masks = _band_masks(W, C)
    bd = sum(
        masks[kw][None, None] * dww_f[:, :, kw, None, :]
        for kw in range(3)
    )
    dwb_f = jnp.tile(dwb, (1, W))
    pwb_f = jnp.tile(pwb, (1, W))
    ch = 256 if WC % 256 == 0 else WC
    reps = ch // C
    pw_c = _pw_mask(reps, C) * jnp.tile(pww, (1, reps, reps))
    return bd, dwb_f, pw_c, pwb_f


def _run(C, n_pw_chunks, Nb, x_f, bd, dwb_f, pw_c, pwb_f):
    N, H, WC = x_f.shape
    n = bd.shape[0]
    ch = pw_c.shape[-1]
    return pl.pallas_call(
        functools.partial(_kernel, C, n_pw_chunks),
        out_shape=jax.ShapeDtypeStruct((N, H, WC), x_f.dtype),
        grid_spec=pltpu.PrefetchScalarGridSpec(
            num_scalar_prefetch=0,
            grid=(N // Nb,),
            in_specs=[
                pl.BlockSpec((Nb, H, WC), lambda b: (b, 0, 0)),
                pl.BlockSpec((n, 3, WC, WC), lambda b: (0, 0, 0, 0)),
                pl.BlockSpec((n, WC), lambda b: (0, 0)),
                pl.BlockSpec((n, ch, ch), lambda b: (0, 0, 0)),
                pl.BlockSpec((n, WC), lambda b: (0, 0)),
            ],
            out_specs=pl.BlockSpec((Nb, H, WC), lambda b: (b, 0, 0)),
            scratch_shapes=[pltpu.VMEM((Nb, H + 2, WC), jnp.float32)],
        ),
        compiler_params=pltpu.CompilerParams(
            dimension_semantics=("parallel",),
            vmem_limit_bytes=64 * 1024 * 1024,
        ),
    )(x_f, bd, dwb_f, pw_c, pwb_f)


def kernel(x_nhwc, dww, dwb, pww, pwb):
    N, H, W, C = x_nhwc.shape
    WC = W * C

    bd, dwb_f, pw_c, pwb_f = _build_params(dww, dwb, pww, pwb, W)
    ch = pw_c.shape[-1]
    n_pw_chunks = WC // ch
    x_f = x_nhwc.reshape(N, H, WC)

    devs = jax.devices()
    n_dev = 2 if (len(devs) >= 2 and N % 2 == 0) else 1
    Nb = next(nb for nb in (16, 8, 4, 2, 1) if (N // n_dev) % nb == 0)
    run = functools.partial(_run, C, n_pw_chunks, Nb)

    if n_dev == 2:
        mesh = jax.sharding.Mesh(np.array(devs[:2]), ("d",))
        p = jax.sharding.PartitionSpec
        run = shard_map(
            run, mesh=mesh,
            in_specs=(p("d"), p(), p(), p(), p()),
            out_specs=p("d"),
            check_rep=False,
        )
    out = run(x_f, bd, dwb_f, pw_c, pwb_f)
    return out.reshape(N, H, W, C)


# speedup vs baseline: 2.2597x; 2.2597x over previous
import functools

import jax
import jax.numpy as jnp
import numpy as np
from jax.experimental import pallas as pl
from jax.experimental.pallas import tpu as pltpu
from jax.experimental.shard_map import shard_map


def _hardswish(x):
    return x * jnp.clip(x + 3.0, 0.0, 6.0) * (1.0 / 6.0)


def _kernel(C, n_pw_chunks, x_ref, bd_ref, dwb_ref, pw_ref, pwb_ref,
            o_ref, xp_ref):
    Nb, H, WC = x_ref.shape
    n_blocks = bd_ref.shape[0]
    CH = pw_ref.shape[-1]

    xp_ref[:, 0:1, :] = jnp.zeros((Nb, 1, WC), jnp.float32)
    xp_ref[:, H + 1:H + 2, :] = jnp.zeros((Nb, 1, WC), jnp.float32)

    x = x_ref[...].astype(jnp.float32).reshape(Nb * H, WC)

    for blk in range(n_blocks):
        xp_ref[:, 1:H + 1, :] = x.reshape(Nb, H, WC)

        a = xp_ref[:, 0:H, :].reshape(Nb * H, WC)
        c = xp_ref[:, 2:H + 2, :].reshape(Nb * H, WC)
        y = (jnp.dot(a, bd_ref[blk, 0], preferred_element_type=jnp.float32)
             + jnp.dot(x, bd_ref[blk, 1], preferred_element_type=jnp.float32)
             + jnp.dot(c, bd_ref[blk, 2], preferred_element_type=jnp.float32))

        y = _hardswish(y + dwb_ref[blk].reshape(1, WC))

        if n_pw_chunks == 1:
            z = jnp.dot(y, pw_ref[blk], preferred_element_type=jnp.float32)
        else:
            z = jnp.concatenate(
                [
                    jnp.dot(y[:, k * CH:(k + 1) * CH], pw_ref[blk],
                            preferred_element_type=jnp.float32)
                    for k in range(n_pw_chunks)
                ],
                axis=1,
            )
        z = _hardswish(z + pwb_ref[blk].reshape(1, WC))

        x = z + x

    o_ref[...] = x.reshape(Nb, H, WC).astype(o_ref.dtype)


def _band_masks(W, C):
    WC = W * C
    masks = np.zeros((3, WC, WC), np.float32)
    eye_c = np.eye(C, dtype=np.float32)
    for kw in range(3):
        for w in range(W):
            v = w + kw - 1
            if 0 <= v < W:
                masks[kw, v * C:(v + 1) * C, w * C:(w + 1) * C] = eye_c
    return masks


def _pw_mask(reps, C):
    m = np.zeros((reps * C, reps * C), np.float32)
    for u in range(reps):
        m[u * C:(u + 1) * C, u * C:(u + 1) * C] = 1.0
    return m


def _build_params(dww, dwb, pww, pwb, W):
    n, _, _, C = dww.shape
    WC = W * C
    dww_f = jnp.tile(dww[:, :, :, None, :], (1, 1, 1, W, 1)).reshape(n, 3, 3, WC)
    w_idx = np.arange(WC) // C
    edge = np.ones((3, WC), np.float32)
    edge[0] = (w_idx >= 1).astype(np.float32)
    edge[2] = (w_idx <= W - 2).astype(np.float32)
    dww_f = dww_f * edge[None, None]
    masks = _band_masks(W, C)
    bd = sum(
        masks[kw][None, None] * dww_f[:, :, kw, None, :]
        for kw in range(3)
    )
    dwb_f = jnp.tile(dwb, (1, W))
    pwb_f = jnp.tile(pwb, (1, W))
    ch = 256 if WC % 256 == 0 else WC
    reps = ch // C
    pw_c = _pw_mask(reps, C) * jnp.tile(pww, (1, reps, reps))
    return bd, dwb_f, pw_c, pwb_f


def _run(C, n_pw_chunks, Nb, x_f, bd, dwb_f, pw_c, pwb_f):
    N, H, WC = x_f.shape
    n = bd.shape[0]
    ch = pw_c.shape[-1]
    return pl.pallas_call(
        functools.partial(_kernel, C, n_pw_chunks),
        out_shape=jax.ShapeDtypeStruct((N, H, WC), x_f.dtype),
        grid_spec=pltpu.PrefetchScalarGridSpec(
            num_scalar_prefetch=0,
            grid=(N // Nb,),
            in_specs=[
                pl.BlockSpec((Nb, H, WC), lambda b: (b, 0, 0)),
                pl.BlockSpec((n, 3, WC, WC), lambda b: (0, 0, 0, 0)),
                pl.BlockSpec((n, WC), lambda b: (0, 0)),
                pl.BlockSpec((n, ch, ch), lambda b: (0, 0, 0)),
                pl.BlockSpec((n, WC), lambda b: (0, 0)),
            ],
            out_specs=pl.BlockSpec((Nb, H, WC), lambda b: (b, 0, 0)),
            scratch_shapes=[pltpu.VMEM((Nb, H + 2, WC), jnp.float32)],
        ),
        compiler_params=pltpu.CompilerParams(
            dimension_semantics=("parallel",),
            vmem_limit_bytes=64 * 1024 * 1024,
        ),
    )(x_f, bd, dwb_f, pw_c, pwb_f)


def kernel(x_nhwc, dww, dwb, pww, pwb):
    N, H, W, C = x_nhwc.shape
    WC = W * C

    bd, dwb_f, pw_c, pwb_f = _build_params(dww, dwb, pww, pwb, W)
    ch = pw_c.shape[-1]
    n_pw_chunks = WC // ch
    x_f = x_nhwc.reshape(N, H, WC)

    devs = jax.devices()
    n_dev = 1
    Nb = next(nb for nb in (16, 8, 4, 2, 1) if (N // n_dev) % nb == 0)
    run = functools.partial(_run, C, n_pw_chunks, Nb)

    if n_dev == 2:
        mesh = jax.sharding.Mesh(np.array(devs[:2]), ("d",))
        p = jax.sharding.PartitionSpec
        run = shard_map(
            run, mesh=mesh,
            in_specs=(p("d"), p(), p(), p(), p()),
            out_specs=p("d"),
            check_rep=False,
        )
    out = run(x_f, bd, dwb_f, pw_c, pwb_f)
    return out.reshape(N, H, W, C)
